# Initial kernel scaffold; baseline (speedup 1.0000x reference)
import jax
import jax.numpy as jnp
from jax.experimental import pallas as pl


def kernel(x, edge_index, W_in, b_in, W_msg1, b_msg1, W_upd1, b_upd1, W_msg2, b_msg2, W_upd2, b_upd2, W_out, b_out):
    raise NotImplementedError("write your pallas kernel here")


# speedup vs baseline: 6.8020x; 6.8020x over previous
import functools

import jax
import jax.numpy as jnp
from jax import lax
from jax.experimental import pallas as pl
from jax.experimental.pallas import tpu as pltpu
from jax.experimental.pallas import tpu_sc as plsc

_NC = 2
_NS = 16
_NW = _NC * _NS
_CH = 80


def _in_msg_body(x_ref, Wi_ref, bi_ref, Wm_ref, bm_ref, h_ref, m_ref):
    h = jnp.maximum(
        jnp.dot(x_ref[...], Wi_ref[...], preferred_element_type=jnp.float32)
        + bi_ref[...], 0.0)
    h_ref[...] = h
    m_ref[...] = jnp.maximum(
        jnp.dot(h, Wm_ref[...], preferred_element_type=jnp.float32)
        + bm_ref[...], 0.0)


def _upd_msg_body(h_ref, p0_ref, p1_ref, Wua_ref, Wub_ref, bu_ref,
                  Wm_ref, bm_ref, h2_ref, m2_ref):
    aggr = p0_ref[0] + p1_ref[0]
    hn = jnp.maximum(
        jnp.dot(h_ref[...], Wua_ref[...], preferred_element_type=jnp.float32)
        + jnp.dot(aggr, Wub_ref[...], preferred_element_type=jnp.float32)
        + bu_ref[...], 0.0)
    h2_ref[...] = hn
    m2_ref[...] = jnp.maximum(
        jnp.dot(hn, Wm_ref[...], preferred_element_type=jnp.float32)
        + bm_ref[...], 0.0)


def _upd_out_body(h_ref, p0_ref, p1_ref, Wua_ref, Wub_ref, bu_ref,
                  wo_ref, bo_ref, out_ref):
    aggr = p0_ref[0] + p1_ref[0]
    hn = jnp.maximum(
        jnp.dot(h_ref[...], Wua_ref[...], preferred_element_type=jnp.float32)
        + jnp.dot(aggr, Wub_ref[...], preferred_element_type=jnp.float32)
        + bu_ref[...], 0.0)
    out_ref[...] = jnp.sum(hn * wo_ref[...], axis=1, keepdims=True) + bo_ref[...]


def _row_spec(br, h):
    return pl.BlockSpec((br, h), lambda i: (i, 0))


def _full_spec(r, c):
    return pl.BlockSpec((r, c), lambda i: (0, 0))


@functools.lru_cache(maxsize=None)
def _make_edge_aggr(n, e, h):
    epw = e // _NW
    nch = epw // _CH
    rpt = n // _NS
    assert epw * _NW == e and nch * _CH == epw and rpt * _NS == n and rpt % 8 == 0

    mesh = plsc.VectorSubcoreMesh(
        core_axis_name="c", subcore_axis_name="s",
        num_cores=_NC, num_subcores=_NS)

    @functools.partial(
        pl.kernel,
        out_type=jax.ShapeDtypeStruct((_NC, n, h), jnp.float32),
        mesh=mesh,
        scratch_types=[
            pltpu.VMEM((nch, _CH), jnp.int32),
            pltpu.VMEM((nch, _CH), jnp.int32),
            pltpu.VMEM((_CH, h), jnp.float32),
            pltpu.VMEM_SHARED((n, h), jnp.float32),
            pltpu.SemaphoreType.DMA,
        ],
    )
    def edge_aggr(m_hbm, src_hbm, dst_hbm, zeros_hbm, out_hbm,
                  sidx, didx, rows, acc, sem):
        c = lax.axis_index("c")
        s = lax.axis_index("s")
        wid = s * _NC + c
        pltpu.sync_copy(zeros_hbm, acc.at[pl.ds(s * rpt, rpt)])
        pltpu.sync_copy(src_hbm.at[wid], sidx)
        pltpu.sync_copy(dst_hbm.at[wid], didx)
        plsc.subcore_barrier()

        def body(j, carry):
            pltpu.async_copy(m_hbm.at[sidx.at[j]], rows, sem).wait()
            pltpu.sync_copy(rows, acc.at[didx.at[j]], add=True)
            return carry

        lax.fori_loop(0, nch, body, 0)
        plsc.subcore_barrier()
        pltpu.sync_copy(acc.at[pl.ds(s * rpt, rpt)],
                        out_hbm.at[c, pl.ds(s * rpt, rpt)])

    return edge_aggr


def kernel(x, edge_index, W_in, b_in, W_msg1, b_msg1, W_upd1, b_upd1,
           W_msg2, b_msg2, W_upd2, b_upd2, W_out, b_out):
    n, d = x.shape
    h = W_in.shape[1]
    e = edge_index.shape[1]
    br = 2000
    grid = (n // br,)

    npad = -(-n // (8 * _NS)) * (8 * _NS)
    src = edge_index[0].astype(jnp.int32).reshape(_NW, e // _NW // _CH, _CH)
    dst = edge_index[1].astype(jnp.int32).reshape(_NW, e // _NW // _CH, _CH)
    zeros = jnp.zeros((npad // _NS, h), jnp.float32)

    bi = b_in.reshape(1, h)
    bm1 = b_msg1.reshape(1, h)
    bu1 = b_upd1.reshape(1, h)
    bm2 = b_msg2.reshape(1, h)
    bu2 = b_upd2.reshape(1, h)
    Wu1a, Wu1b = W_upd1[:h], W_upd1[h:]
    Wu2a, Wu2b = W_upd2[:h], W_upd2[h:]
    wo = W_out.reshape(1, h)
    bo = b_out.reshape(1, 1)

    edge_aggr = _make_edge_aggr(npad, e, h)
    p_spec0 = pl.BlockSpec((1, br, h), lambda i: (0, i, 0))
    p_spec1 = pl.BlockSpec((1, br, h), lambda i: (1, i, 0))

    h0, m1 = pl.pallas_call(
        _in_msg_body,
        grid=grid,
        in_specs=[_row_spec(br, d), _full_spec(d, h), _full_spec(1, h),
                  _full_spec(h, h), _full_spec(1, h)],
        out_specs=[_row_spec(br, h), _row_spec(br, h)],
        out_shape=[jax.ShapeDtypeStruct((n, h), jnp.float32),
                   jax.ShapeDtypeStruct((n, h), jnp.float32)],
    )(x, W_in, bi, W_msg1, bm1)

    p1 = edge_aggr(m1, src, dst, zeros)

    h1, m2 = pl.pallas_call(
        _upd_msg_body,
        grid=grid,
        in_specs=[_row_spec(br, h), p_spec0, p_spec1,
                  _full_spec(h, h), _full_spec(h, h), _full_spec(1, h),
                  _full_spec(h, h), _full_spec(1, h)],
        out_specs=[_row_spec(br, h), _row_spec(br, h)],
        out_shape=[jax.ShapeDtypeStruct((n, h), jnp.float32),
                   jax.ShapeDtypeStruct((n, h), jnp.float32)],
    )(h0, p1, p1, Wu1a, Wu1b, bu1, W_msg2, bm2)

    p2 = edge_aggr(m2, src, dst, zeros)

    out = pl.pallas_call(
        _upd_out_body,
        grid=grid,
        in_specs=[_row_spec(br, h), p_spec0, p_spec1,
                  _full_spec(h, h), _full_spec(h, h), _full_spec(1, h),
                  _full_spec(1, h), _full_spec(1, 1)],
        out_specs=_row_spec(br, 1),
        out_shape=jax.ShapeDtypeStruct((n, 1), jnp.float32),
    )(h1, p2, p2, Wu2a, Wu2b, bu2, wo, bo)

    return out.reshape(n)


# speedup vs baseline: 10.5146x; 1.5458x over previous
import functools

import jax
import jax.numpy as jnp
from jax import lax
from jax.experimental import pallas as pl
from jax.experimental.pallas import tpu as pltpu
from jax.experimental.pallas import tpu_sc as plsc

_NC = 2
_NS = 16
_NW = _NC * _NS
_CH = 80


def _in_msg_body(x_ref, Wi_ref, bi_ref, Wm_ref, bm_ref, h_ref, m_ref):
    h = jnp.maximum(
        jnp.dot(x_ref[...], Wi_ref[...], preferred_element_type=jnp.float32)
        + bi_ref[...], 0.0)
    h_ref[...] = h
    m_ref[...] = jnp.maximum(
        jnp.dot(h, Wm_ref[...], preferred_element_type=jnp.float32)
        + bm_ref[...], 0.0)


def _upd_msg_body(h_ref, p0_ref, p1_ref, Wu_ref, bu_ref,
                  Wm_ref, bm_ref, h2_ref, m2_ref):
    cat = jnp.concatenate([h_ref[...], p0_ref[0] + p1_ref[0]], axis=-1)
    hn = jnp.maximum(
        jnp.dot(cat, Wu_ref[...], preferred_element_type=jnp.float32)
        + bu_ref[...], 0.0)
    h2_ref[...] = hn
    m2_ref[...] = jnp.maximum(
        jnp.dot(hn, Wm_ref[...], preferred_element_type=jnp.float32)
        + bm_ref[...], 0.0)


def _upd_out_body(h_ref, p0_ref, p1_ref, Wu_ref, bu_ref,
                  wo_ref, bo_ref, out_ref):
    cat = jnp.concatenate([h_ref[...], p0_ref[0] + p1_ref[0]], axis=-1)
    hn = jnp.maximum(
        jnp.dot(cat, Wu_ref[...], preferred_element_type=jnp.float32)
        + bu_ref[...], 0.0)
    out_ref[...] = (jnp.dot(hn, wo_ref[...], preferred_element_type=jnp.float32)
                    + bo_ref[...])


def _row_spec(br, h):
    return pl.BlockSpec((br, h), lambda i: (i, 0))


def _full_spec(r, c):
    return pl.BlockSpec((r, c), lambda i: (0, 0))


@functools.lru_cache(maxsize=None)
def _make_edge_aggr(n, e, h):
    epw = e // _NW
    nch = epw // _CH
    rpt = n // _NS
    assert epw * _NW == e and nch * _CH == epw and rpt * _NS == n and rpt % 8 == 0
    assert nch % 2 == 1

    mesh = plsc.VectorSubcoreMesh(
        core_axis_name="c", subcore_axis_name="s",
        num_cores=_NC, num_subcores=_NS)

    @functools.partial(
        pl.kernel,
        out_type=jax.ShapeDtypeStruct((_NC, n, h), jnp.float32),
        mesh=mesh,
        scratch_types=[
            pltpu.VMEM((_CH,), jnp.int32),
            pltpu.VMEM((_CH,), jnp.int32),
            pltpu.VMEM((nch, _CH), jnp.int32),
            pltpu.VMEM((_CH, h), jnp.float32),
            pltpu.VMEM((_CH, h), jnp.float32),
            pltpu.VMEM_SHARED((n, h), jnp.float32),
            pltpu.SemaphoreType.DMA,
            pltpu.SemaphoreType.DMA,
            pltpu.SemaphoreType.DMA,
            pltpu.SemaphoreType.DMA,
        ],
    )
    def edge_aggr(m_hbm, src_hbm, dst_hbm, zeros_hbm, out_hbm,
                  siA, siB, didx, rows0, rows1, acc, g0, g1, isA, isB):
        c = lax.axis_index("c")
        s = lax.axis_index("s")
        wid = s * _NC + c
        base = wid * epw
        pltpu.sync_copy(zeros_hbm, acc.at[pl.ds(s * rpt, rpt)])
        pltpu.sync_copy(dst_hbm.at[wid], didx)
        plsc.subcore_barrier()

        def iload(j, buf, sem):
            pltpu.async_copy(src_hbm.at[pl.ds(base + j * _CH, _CH)], buf, sem)

        iload(0, siA, isA)
        iload(1, siB, isB)
        pltpu.make_async_copy(src_hbm.at[pl.ds(base, _CH)], siA, isA).wait()
        pltpu.async_copy(m_hbm.at[siA], rows0, g0)

        def body(t, carry):
            j0 = 2 * t
            pltpu.make_async_copy(src_hbm.at[pl.ds(base, _CH)], siB, isB).wait()
            pltpu.async_copy(m_hbm.at[siB], rows1, g1)
            pltpu.make_async_copy(m_hbm.at[siA], rows0, g0).wait()
            iload(j0 + 2, siA, isA)
            pltpu.sync_copy(rows0, acc.at[didx.at[j0]], add=True)
            pltpu.make_async_copy(src_hbm.at[pl.ds(base, _CH)], siA, isA).wait()
            pltpu.async_copy(m_hbm.at[siA], rows0, g0)
            pltpu.make_async_copy(m_hbm.at[siB], rows1, g1).wait()
            iload(jnp.minimum(j0 + 3, nch - 1), siB, isB)
            pltpu.sync_copy(rows1, acc.at[didx.at[j0 + 1]], add=True)
            return carry

        lax.fori_loop(0, (nch - 1) // 2, body, 0)
        pltpu.make_async_copy(src_hbm.at[pl.ds(base, _CH)], siB, isB).wait()
        pltpu.make_async_copy(m_hbm.at[siA], rows0, g0).wait()
        pltpu.sync_copy(rows0, acc.at[didx.at[nch - 1]], add=True)
        plsc.subcore_barrier()
        pltpu.sync_copy(acc.at[pl.ds(s * rpt, rpt)],
                        out_hbm.at[c, pl.ds(s * rpt, rpt)])

    return edge_aggr


def kernel(x, edge_index, W_in, b_in, W_msg1, b_msg1, W_upd1, b_upd1,
           W_msg2, b_msg2, W_upd2, b_upd2, W_out, b_out):
    n, d = x.shape
    h = W_in.shape[1]
    e = edge_index.shape[1]
    br = 2000
    grid = (n // br,)

    npad = -(-n // (8 * _NS)) * (8 * _NS)
    src = edge_index[0].astype(jnp.int32)
    dst = edge_index[1].astype(jnp.int32).reshape(_NW, e // _NW // _CH, _CH)
    zeros = jnp.zeros((npad // _NS, h), jnp.float32)

    bi = b_in.reshape(1, h)
    bm1 = b_msg1.reshape(1, h)
    bu1 = b_upd1.reshape(1, h)
    bm2 = b_msg2.reshape(1, h)
    bu2 = b_upd2.reshape(1, h)
    bo = b_out.reshape(1, 1)

    edge_aggr = _make_edge_aggr(npad, e, h)
    p_spec0 = pl.BlockSpec((1, br, h), lambda i: (0, i, 0))
    p_spec1 = pl.BlockSpec((1, br, h), lambda i: (1, i, 0))

    h0, m1 = pl.pallas_call(
        _in_msg_body,
        grid=grid,
        in_specs=[_row_spec(br, d), _full_spec(d, h), _full_spec(1, h),
                  _full_spec(h, h), _full_spec(1, h)],
        out_specs=[_row_spec(br, h), _row_spec(br, h)],
        out_shape=[jax.ShapeDtypeStruct((n, h), jnp.float32),
                   jax.ShapeDtypeStruct((n, h), jnp.float32)],
    )(x, W_in, bi, W_msg1, bm1)

    p1 = edge_aggr(m1, src, dst, zeros)

    h1, m2 = pl.pallas_call(
        _upd_msg_body,
        grid=grid,
        in_specs=[_row_spec(br, h), p_spec0, p_spec1,
                  _full_spec(2 * h, h), _full_spec(1, h),
                  _full_spec(h, h), _full_spec(1, h)],
        out_specs=[_row_spec(br, h), _row_spec(br, h)],
        out_shape=[jax.ShapeDtypeStruct((n, h), jnp.float32),
                   jax.ShapeDtypeStruct((n, h), jnp.float32)],
    )(h0, p1, p1, W_upd1, bu1, W_msg2, bm2)

    p2 = edge_aggr(m2, src, dst, zeros)

    out = pl.pallas_call(
        _upd_out_body,
        grid=grid,
        in_specs=[_row_spec(br, h), p_spec0, p_spec1,
                  _full_spec(2 * h, h), _full_spec(1, h),
                  _full_spec(h, 1), _full_spec(1, 1)],
        out_specs=_row_spec(br, 1),
        out_shape=jax.ShapeDtypeStruct((n, 1), jnp.float32),
    )(h1, p2, p2, W_upd2, bu2, W_out, bo)

    return out.reshape(n)
